# baseline (speedup 1.0000x reference)
import jax
import jax.numpy as jnp
from jax import lax
from jax.experimental import pallas as pl
from jax.experimental.pallas import tpu as pltpu
from jax.experimental.pallas import tpu_sc as plsc

_S = 13
_CELLS = _S * _S
_T = 20
_TP = 104
_L = 16
_NW = 32
_BATCH = 128
_BPW = _BATCH // _NW
_CONF1 = 18


def _body(preds_hbm, c2_hbm, tg_hbm, out_hbm,
          tg_v, blk_v, b2_v, cellbuf, acc_v, sems):
    wid = lax.axis_index("s") * 2 + lax.axis_index("c")
    lanes = lax.iota(jnp.int32, _L)
    zeros = jnp.zeros((_L,), jnp.float32)

    def splat(v):
        return jnp.full((_L,), v, jnp.int32)

    b0 = wid * _BPW
    copies = []
    for i in range(_BPW):
        b = b0 + i
        copies.append((
            pltpu.async_copy(preds_hbm.at[b, 0:2, 0],
                             blk_v.at[i, 0:2, 0], sems.at[i, 0]),
            pltpu.async_copy(preds_hbm.at[b, 0:2, 1, :, 0:48],
                             blk_v.at[i, 0:2, 1, :, 0:48], sems.at[i, 1]),
            pltpu.async_copy(preds_hbm.at[b, 2, 0, 0:3],
                             blk_v.at[i, 2, 0, 0:3], sems.at[i, 2]),
            pltpu.async_copy(preds_hbm.at[b, 2, 1, 0:3, 0:48],
                             blk_v.at[i, 2, 1, 0:3, 0:48], sems.at[i, 3]),
        ))
    pltpu.sync_copy(tg_hbm.at[pl.ds(b0, _BPW)], tg_v)

    c0 = 5 * wid + jnp.minimum(wid, 9)
    pltpu.sync_copy(c2_hbm.at[pl.ds(c0, 6)], b2_v)
    sixth = lax.broadcast_in_dim(wid < 9, (_L,), ())
    acc_c = zeros
    for cc in range(6):
        ssq = zeros
        for jj in range(8):
            v = b2_v[cc, pl.ds(jj * _L, _L)]
            ssq = ssq + v * v
        if cc == 5:
            ssq = jnp.where(sixth, ssq, 0.0)
        acc_c = acc_c + ssq

    acc_m = zeros

    tail9 = lanes < 9

    for i in range(_BPW):
        for c in copies[i]:
            c.wait()
        isp = splat(i)

        def pick(ch, hv, jv):
            return plsc.load_gather(
                blk_v, [isp, splat(ch // 8), hv, splat(ch % 8), jv])

        per_chunk = []
        for chunk in range(2):
            tvec = lanes + chunk * _L
            fidx = jnp.minimum(tvec, _T - 1) * 5

            def field(f):
                return plsc.load_gather(tg_v, [isp, fidx + f])

            cls = field(0)
            cx = field(1)
            cy = field(2)
            w = field(3)
            h = field(4)

            gx = (cx * _S).astype(jnp.int32)
            gy = (cy * _S).astype(jnp.int32)
            valid = (gx < _S) & (gy < _S) & (tvec < _T)
            gxc = jnp.clip(gx, 0, _S - 1)
            gyc = jnp.clip(gy, 0, _S - 1)
            cell = gyc * _S + gxc
            hv = lax.shift_right_logical(cell, 7)
            jv = cell & 127

            d1 = pick(1, hv, jv) - cx
            d2 = pick(2, hv, jv) - cy
            d3 = pick(3, hv, jv) - w
            d4 = pick(4, hv, jv) - h
            coord = d1 * d1 + d2 * d2 + d3 * d3 + d4 * d4

            k = cls.astype(jnp.int32)
            cls_l = zeros
            for c in range(13):
                p = pick(5 + c, hv, jv)
                d = jnp.where(k == c, p - 1.0, p)
                cls_l = cls_l + d * d

            contrib = 5.0 * coord + cls_l
            acc_m = acc_m + jnp.where(valid, contrib, 0.0)

            plsc.store_scatter(cellbuf, [cell], tvec, mask=valid)
            per_chunk.append((tvec, cell, hv, jv, valid))

        for tvec, cell, hv, jv, valid in per_chunk:
            rb = plsc.load_gather(cellbuf, [cell])
            winner = valid & (rb == tvec)
            c0t = pick(0, hv, jv)
            acc_c = acc_c - jnp.where(winner, c0t * c0t, 0.0)

        for ch in (0, _CONF1):
            for hh, jj in [(0, j) for j in range(8)] + [(1, j) for j in range(3)]:
                v = blk_v[i, ch // 8, hh, ch % 8, pl.ds(jj * _L, _L)]
                if hh == 1 and jj == 2:
                    v = jnp.where(tail9, v, 0.0)
                acc_c = acc_c + v * v

    acc_v[...] = acc_m + 0.5 * acc_c
    pltpu.sync_copy(acc_v, out_hbm.at[pl.ds(wid * _L, _L)])


def kernel(predictions, targets):
    pA = jnp.pad(
        predictions[:, :24].reshape(_BATCH, 24, _CELLS),
        ((0, 0), (0, 0), (0, 256 - _CELLS)),
    ).reshape(_BATCH, 3, 8, 2, 128).transpose(0, 1, 3, 2, 4)
    pB = jnp.pad(
        jnp.transpose(predictions[:, 36:37], (2, 3, 1, 0)).reshape(_CELLS, _BATCH),
        ((0, 176 - _CELLS), (0, 0)),
    )
    tg2 = jnp.pad(targets.reshape(_BATCH, 5 * _T), ((0, 0), (0, _TP - 5 * _T)))
    mesh = plsc.VectorSubcoreMesh(
        core_axis_name="c", subcore_axis_name="s", num_cores=2, num_subcores=16)
    out = pl.kernel(
        _body,
        out_type=jax.ShapeDtypeStruct((_NW * _L,), jnp.float32),
        mesh=mesh,
        compiler_params=pltpu.CompilerParams(
            use_tc_tiling_on_sc=False, needs_layout_passes=False),
        scratch_types=[
            pltpu.VMEM((_BPW, _TP), jnp.float32),
            pltpu.VMEM((_BPW, 3, 2, 8, 128), jnp.float32),
            pltpu.VMEM((6, 128), jnp.float32),
            pltpu.VMEM((_CELLS,), jnp.int32),
            pltpu.VMEM((_L,), jnp.float32),
            pltpu.SemaphoreType.DMA((_BPW, 4)),
        ],
    )(pA, pB, tg2)
    return jnp.sum(out) / _BATCH


# speedup vs baseline: 1.0295x; 1.0295x over previous
import jax
import jax.numpy as jnp
from jax import lax
from jax.experimental import pallas as pl
from jax.experimental.pallas import tpu as pltpu
from jax.experimental.pallas import tpu_sc as plsc

_S = 13
_CELLS = _S * _S
_T = 20
_TP = 104
_L = 16
_NW = 32
_BATCH = 128
_BPW = _BATCH // _NW
_CONF1 = 18
_CONF2 = 23


def _body(preds_hbm, tg_hbm, out_hbm, tg_v, blk_v, cellbuf, acc_v):
    wid = lax.axis_index("s") * 2 + lax.axis_index("c")
    lanes = lax.iota(jnp.int32, _L)
    zeros = jnp.zeros((_L,), jnp.float32)

    def splat(v):
        return jnp.full((_L,), v, jnp.int32)

    b0 = wid * _BPW
    pltpu.sync_copy(preds_hbm.at[pl.ds(b0, _BPW)], blk_v)
    pltpu.sync_copy(tg_hbm.at[pl.ds(b0, _BPW)], tg_v)

    acc_m = zeros
    acc_c = zeros

    for i in range(_BPW):
        isp = splat(i)

        def pick(ch, hv, jv):
            return plsc.load_gather(
                blk_v, [isp, splat(ch // 8), hv, splat(ch % 8), jv])

        per_chunk = []
        for chunk in range(2):
            tvec = lanes + chunk * _L
            fidx = jnp.minimum(tvec, _T - 1) * 5

            def field(f):
                return plsc.load_gather(tg_v, [isp, fidx + f])

            cls = field(0)
            cx = field(1)
            cy = field(2)
            w = field(3)
            h = field(4)

            gx = (cx * _S).astype(jnp.int32)
            gy = (cy * _S).astype(jnp.int32)
            valid = (gx < _S) & (gy < _S) & (tvec < _T)
            gxc = jnp.clip(gx, 0, _S - 1)
            gyc = jnp.clip(gy, 0, _S - 1)
            cell = gyc * _S + gxc
            hv = lax.shift_right_logical(cell, 7)
            jv = cell & 127

            d1 = pick(1, hv, jv) - cx
            d2 = pick(2, hv, jv) - cy
            d3 = pick(3, hv, jv) - w
            d4 = pick(4, hv, jv) - h
            coord = d1 * d1 + d2 * d2 + d3 * d3 + d4 * d4

            k = cls.astype(jnp.int32)
            cls_l = zeros
            for c in range(13):
                p = pick(5 + c, hv, jv)
                d = jnp.where(k == c, p - 1.0, p)
                cls_l = cls_l + d * d

            contrib = 5.0 * coord + cls_l
            acc_m = acc_m + jnp.where(valid, contrib, 0.0)

            plsc.store_scatter(cellbuf, [cell], tvec, mask=valid)
            per_chunk.append((tvec, cell, hv, jv, valid))

        for tvec, cell, hv, jv, valid in per_chunk:
            rb = plsc.load_gather(cellbuf, [cell])
            winner = valid & (rb == tvec)
            c0t = pick(0, hv, jv)
            acc_c = acc_c - jnp.where(winner, c0t * c0t, 0.0)

        for ch in (0, _CONF1, _CONF2):
            for hh in range(2):
                for jj in range(8):
                    v = blk_v[i, ch // 8, hh, ch % 8, pl.ds(jj * _L, _L)]
                    acc_c = acc_c + v * v

    acc_v[...] = acc_m + 0.5 * acc_c
    pltpu.sync_copy(acc_v, out_hbm.at[wid])


def kernel(predictions, targets):
    p24 = jnp.concatenate(
        [predictions[:, :23], predictions[:, 36:37]], axis=1
    ).reshape(_BATCH, 24, _CELLS)
    p24 = jnp.pad(p24, ((0, 0), (0, 0), (0, 256 - _CELLS)))
    p24 = p24.reshape(_BATCH, 3, 8, 2, 128).transpose(0, 1, 3, 2, 4)
    tg2 = jnp.pad(targets.reshape(_BATCH, 5 * _T), ((0, 0), (0, _TP - 5 * _T)))
    mesh = plsc.VectorSubcoreMesh(
        core_axis_name="c", subcore_axis_name="s", num_cores=2, num_subcores=16)
    out = pl.kernel(
        _body,
        out_type=jax.ShapeDtypeStruct((_NW, _L), jnp.float32),
        mesh=mesh,
        compiler_params=pltpu.CompilerParams(
            use_tc_tiling_on_sc=False, needs_layout_passes=False),
        scratch_types=[
            pltpu.VMEM((_BPW, _TP), jnp.float32),
            pltpu.VMEM((_BPW, 3, 2, 8, 128), jnp.float32),
            pltpu.VMEM((_CELLS,), jnp.int32),
            pltpu.VMEM((_L,), jnp.float32),
        ],
    )(p24, tg2)
    return jnp.sum(out) / _BATCH
